# baseline (speedup 1.0000x reference)
import functools

import jax
import jax.numpy as jnp
from jax.experimental import pallas as pl

NPG = 61
P = 2
G = 32
PR = 2 * NPG


def _body(x_ref, we_ref, be_ref, wp_ref, bp_ref, wc_ref, bc_ref,
          out_ref, adj_ref):
    xb = x_ref[...]
    z = jnp.dot(xb, we_ref[...], preferred_element_type=jnp.float32)
    z = jnp.maximum(z + be_ref[...], 0.0)

    z8 = z * 0.125

    rows = jax.lax.broadcasted_iota(jnp.int32, (PR, PR), 0)
    cols = jax.lax.broadcasted_iota(jnp.int32, (PR, PR), 1)
    cross = (rows // NPG) != (cols // NPG)
    dead = cross | (rows == cols)
    neg = jnp.float32(-1e30)
    sel = ((jax.lax.broadcasted_iota(jnp.int32, (P, PR), 1) // NPG)
           == jax.lax.broadcasted_iota(jnp.int32, (P, PR), 0)
           ).astype(jnp.float32)

    hg_rows = []
    for p in range(G // P):
        r0 = p * PR
        zp = jax.lax.slice(z8, (r0, 0), (r0 + PR, z8.shape[1]))
        zq = jax.lax.slice(z, (r0, 0), (r0 + PR, z.shape[1]))
        xp_ = jax.lax.slice(xb, (r0, 0), (r0 + PR, xb.shape[1]))
        logits = jax.lax.dot_general(
            zp, zq, (((1,), (1,)), ((), ())),
            preferred_element_type=jnp.float32)
        logits = jnp.where(cross, neg, logits)
        m1 = jnp.max(logits, axis=1, keepdims=True)
        e = jnp.exp(logits - m1)
        s1 = jnp.sum(e, axis=1, keepdims=True)
        adj = e / s1
        adj_ref[pl.ds(r0, NPG), :] = jax.lax.slice(
            adj, (0, 0), (NPG, NPG))
        adj_ref[pl.ds(r0 + NPG, NPG), :] = jax.lax.slice(
            adj, (NPG, NPG), (PR, PR))
        m0 = jnp.max(logits, axis=0, keepdims=True)
        e2 = jnp.exp(logits - m0)
        s0 = jnp.sum(e2, axis=0, keepdims=True)
        at_full = e2 / s0
        mask = 1.0 + jnp.sum(at_full, axis=1, keepdims=True)
        xm = xp_ * mask
        a0t = jnp.where(dead, 0.0, at_full)
        agg = jnp.dot(a0t, xm,
                      preferred_element_type=jnp.float32)
        h = jnp.dot(agg, wp_ref[...], preferred_element_type=jnp.float32)
        h = jnp.maximum(h + bp_ref[...], 0.0)
        hg_rows.append(jnp.dot(sel, h,
                               preferred_element_type=jnp.float32))

    hg = jnp.concatenate(hg_rows, axis=0)
    out_ref[...] = (
        jnp.dot(hg, wc_ref[...], preferred_element_type=jnp.float32)
        + bc_ref[...])


@functools.partial(jax.jit, static_argnames=("interpret",))
def _run(x, W_enc, b_enc, W_proc, b_proc, W_cls, b_cls, interpret=False):
    N, D = x.shape
    Bv = N // NPG
    H = W_enc.shape[1]
    C = W_cls.shape[1]
    grid = (Bv // G,)
    blk = G * NPG

    out, adj_flat = pl.pallas_call(
        _body,
        grid=grid,
        in_specs=[
            pl.BlockSpec((blk, D), lambda i: (i, 0)),
            pl.BlockSpec((D, H), lambda i: (0, 0)),
            pl.BlockSpec((1, H), lambda i: (0, 0)),
            pl.BlockSpec((D, H), lambda i: (0, 0)),
            pl.BlockSpec((1, H), lambda i: (0, 0)),
            pl.BlockSpec((H, C), lambda i: (0, 0)),
            pl.BlockSpec((1, C), lambda i: (0, 0)),
        ],
        out_specs=[
            pl.BlockSpec((G, C), lambda i: (i, 0)),
            pl.BlockSpec((blk, NPG), lambda i: (i, 0)),
        ],
        out_shape=[
            jax.ShapeDtypeStruct((Bv, C), jnp.float32),
            jax.ShapeDtypeStruct((N, NPG), jnp.float32),
        ],
        interpret=interpret,
    )(x, W_enc, b_enc.reshape(1, H), W_proc, b_proc.reshape(1, H),
      W_cls, b_cls.reshape(1, C))

    return out, adj_flat.reshape(Bv, NPG, NPG)


def kernel(x, edge_index, batch, W_enc, b_enc, W_proc, b_proc, W_cls, b_cls):
    return _run(x, W_enc, b_enc, W_proc, b_proc, W_cls, b_cls)


# speedup vs baseline: 1.4137x; 1.4137x over previous
import functools

import jax
import jax.numpy as jnp
from jax.experimental import pallas as pl

NPG = 61
P = 2
G = 32
PR = 2 * NPG


def _body(x_ref, we_ref, be_ref, wp_ref, bp_ref, wc_ref, bc_ref,
          out_ref, adj_ref):
    xb = x_ref[...]
    z = jnp.dot(xb, we_ref[...], preferred_element_type=jnp.float32)
    z = jnp.maximum(z + be_ref[...], 0.0)

    z8 = z * 0.125

    rows = jax.lax.broadcasted_iota(jnp.int32, (PR, PR), 0)
    cols = jax.lax.broadcasted_iota(jnp.int32, (PR, PR), 1)
    cross = (rows // NPG) != (cols // NPG)
    dead = cross | (rows == cols)
    neg = jnp.float32(-1e30)

    hg_rows = []
    for p in range(G // P):
        r0 = p * PR
        zp = jax.lax.slice(z8, (r0, 0), (r0 + PR, z8.shape[1]))
        zq = jax.lax.slice(z, (r0, 0), (r0 + PR, z.shape[1]))
        xp_ = jax.lax.slice(xb, (r0, 0), (r0 + PR, xb.shape[1]))
        logits = jax.lax.dot_general(
            zp, zq, (((1,), (1,)), ((), ())),
            preferred_element_type=jnp.float32)
        logits = jnp.where(cross, neg, logits)
        m1 = jnp.max(logits, axis=1, keepdims=True)
        e = jnp.exp(logits - m1)
        s1 = jnp.sum(e, axis=1, keepdims=True)
        adj = e / s1
        adj_ref[pl.ds(r0, NPG), :] = jax.lax.slice(
            adj, (0, 0), (NPG, NPG))
        adj_ref[pl.ds(r0 + NPG, NPG), :] = jax.lax.slice(
            adj, (NPG, NPG), (PR, PR))
        m0 = jnp.max(logits, axis=0, keepdims=True)
        e2 = jnp.exp(logits - m0)
        s0 = jnp.sum(e2, axis=0, keepdims=True)
        at_full = e2 / s0
        mask = 1.0 + jnp.sum(at_full, axis=1, keepdims=True)
        xm = xp_ * mask
        a0t = jnp.where(dead, 0.0, at_full)
        agg = jnp.dot(a0t, xm,
                      preferred_element_type=jnp.float32)
        h = jnp.dot(agg, wp_ref[...], preferred_element_type=jnp.float32)
        h = jnp.maximum(h + bp_ref[...], 0.0)
        hg_rows.append(jnp.sum(
            jax.lax.slice(h, (0, 0), (NPG, h.shape[1])), axis=0,
            keepdims=True))
        hg_rows.append(jnp.sum(
            jax.lax.slice(h, (NPG, 0), (PR, h.shape[1])), axis=0,
            keepdims=True))

    hg = jnp.concatenate(hg_rows, axis=0)
    out_ref[...] = (
        jnp.dot(hg, wc_ref[...], preferred_element_type=jnp.float32)
        + bc_ref[...])


@functools.partial(jax.jit, static_argnames=("interpret",))
def _run(x, W_enc, b_enc, W_proc, b_proc, W_cls, b_cls, interpret=False):
    N, D = x.shape
    Bv = N // NPG
    H = W_enc.shape[1]
    C = W_cls.shape[1]
    grid = (Bv // G,)
    blk = G * NPG

    out, adj_flat = pl.pallas_call(
        _body,
        grid=grid,
        in_specs=[
            pl.BlockSpec((blk, D), lambda i: (i, 0)),
            pl.BlockSpec((D, H), lambda i: (0, 0)),
            pl.BlockSpec((1, H), lambda i: (0, 0)),
            pl.BlockSpec((D, H), lambda i: (0, 0)),
            pl.BlockSpec((1, H), lambda i: (0, 0)),
            pl.BlockSpec((H, C), lambda i: (0, 0)),
            pl.BlockSpec((1, C), lambda i: (0, 0)),
        ],
        out_specs=[
            pl.BlockSpec((G, C), lambda i: (i, 0)),
            pl.BlockSpec((blk, NPG), lambda i: (i, 0)),
        ],
        out_shape=[
            jax.ShapeDtypeStruct((Bv, C), jnp.float32),
            jax.ShapeDtypeStruct((N, NPG), jnp.float32),
        ],
        interpret=interpret,
    )(x, W_enc, b_enc.reshape(1, H), W_proc, b_proc.reshape(1, H),
      W_cls, b_cls.reshape(1, C))

    return out, adj_flat.reshape(Bv, NPG, NPG)


def kernel(x, edge_index, batch, W_enc, b_enc, W_proc, b_proc, W_cls, b_cls):
    return _run(x, W_enc, b_enc, W_proc, b_proc, W_cls, b_cls)


# speedup vs baseline: 1.6264x; 1.1504x over previous
import functools

import jax
import jax.numpy as jnp
from jax.experimental import pallas as pl

NPG = 61
P = 2
G = 32
PR = 2 * NPG


def _body(x_ref, we_ref, be_ref, wp_ref, bp_ref, wc_ref, bc_ref,
          out_ref, adj_ref):
    xb = x_ref[...]
    z = jnp.dot(xb, we_ref[...], preferred_element_type=jnp.float32)
    z = jnp.maximum(z + be_ref[...], 0.0)

    z8 = z * 0.125

    rows = jax.lax.broadcasted_iota(jnp.int32, (PR, PR), 0)
    cols = jax.lax.broadcasted_iota(jnp.int32, (PR, PR), 1)
    cross = (rows // NPG) != (cols // NPG)
    dead = cross | (rows == cols)
    neg = jnp.float32(-1e30)

    hg_rows = []
    for p in range(G // P):
        r0 = p * PR
        zp = jax.lax.slice(z8, (r0, 0), (r0 + PR, z8.shape[1]))
        zq = jax.lax.slice(z, (r0, 0), (r0 + PR, z.shape[1]))
        xp_ = jax.lax.slice(xb, (r0, 0), (r0 + PR, xb.shape[1]))
        logits = jax.lax.dot_general(
            zp, zq, (((1,), (1,)), ((), ())),
            preferred_element_type=jnp.float32)
        logits = jnp.where(cross, neg, logits)
        e = jnp.exp(logits)
        s1 = jnp.sum(e, axis=1, keepdims=True)
        adj = e / s1
        adj_ref[pl.ds(r0, NPG), :] = jax.lax.slice(
            adj, (0, 0), (NPG, NPG))
        adj_ref[pl.ds(r0 + NPG, NPG), :] = jax.lax.slice(
            adj, (NPG, NPG), (PR, PR))
        s0 = jnp.sum(e, axis=0, keepdims=True)
        at_full = e / s0
        mask = 1.0 + jnp.sum(at_full, axis=1, keepdims=True)
        xm = xp_ * mask
        a0t = jnp.where(dead, 0.0, at_full)
        agg = jnp.dot(a0t, xm,
                      preferred_element_type=jnp.float32)
        h = jnp.dot(agg, wp_ref[...], preferred_element_type=jnp.float32)
        h = jnp.maximum(h + bp_ref[...], 0.0)
        hg_rows.append(jnp.sum(
            jax.lax.slice(h, (0, 0), (NPG, h.shape[1])), axis=0,
            keepdims=True))
        hg_rows.append(jnp.sum(
            jax.lax.slice(h, (NPG, 0), (PR, h.shape[1])), axis=0,
            keepdims=True))

    hg = jnp.concatenate(hg_rows, axis=0)
    out_ref[...] = (
        jnp.dot(hg, wc_ref[...], preferred_element_type=jnp.float32)
        + bc_ref[...])


@functools.partial(jax.jit, static_argnames=("interpret",))
def _run(x, W_enc, b_enc, W_proc, b_proc, W_cls, b_cls, interpret=False):
    N, D = x.shape
    Bv = N // NPG
    H = W_enc.shape[1]
    C = W_cls.shape[1]
    grid = (Bv // G,)
    blk = G * NPG

    out, adj_flat = pl.pallas_call(
        _body,
        grid=grid,
        in_specs=[
            pl.BlockSpec((blk, D), lambda i: (i, 0)),
            pl.BlockSpec((D, H), lambda i: (0, 0)),
            pl.BlockSpec((1, H), lambda i: (0, 0)),
            pl.BlockSpec((D, H), lambda i: (0, 0)),
            pl.BlockSpec((1, H), lambda i: (0, 0)),
            pl.BlockSpec((H, C), lambda i: (0, 0)),
            pl.BlockSpec((1, C), lambda i: (0, 0)),
        ],
        out_specs=[
            pl.BlockSpec((G, C), lambda i: (i, 0)),
            pl.BlockSpec((blk, NPG), lambda i: (i, 0)),
        ],
        out_shape=[
            jax.ShapeDtypeStruct((Bv, C), jnp.float32),
            jax.ShapeDtypeStruct((N, NPG), jnp.float32),
        ],
        interpret=interpret,
    )(x, W_enc, b_enc.reshape(1, H), W_proc, b_proc.reshape(1, H),
      W_cls, b_cls.reshape(1, C))

    return out, adj_flat.reshape(Bv, NPG, NPG)


def kernel(x, edge_index, batch, W_enc, b_enc, W_proc, b_proc, W_cls, b_cls):
    return _run(x, W_enc, b_enc, W_proc, b_proc, W_cls, b_cls)


# speedup vs baseline: 1.7355x; 1.0671x over previous
import functools

import jax
import jax.numpy as jnp
from jax.experimental import pallas as pl

NPG = 61
P = 2
G = 32
PR = 2 * NPG


def _body(x_ref, we_ref, be_ref, wp_ref, bp_ref, wc_ref, bc_ref,
          out_ref, adj_ref):
    xb = x_ref[...]
    z = jnp.dot(xb, we_ref[...], preferred_element_type=jnp.float32)
    z = jnp.maximum(z + be_ref[...], 0.0)

    z8 = z * 0.125

    rows = jax.lax.broadcasted_iota(jnp.int32, (PR, PR), 0)
    cols = jax.lax.broadcasted_iota(jnp.int32, (PR, PR), 1)
    cross = (rows // NPG) != (cols // NPG)
    dead = cross | (rows == cols)
    neg = jnp.float32(-1e30)

    hg_rows = []
    for p in range(G // P):
        r0 = p * PR
        zp = jax.lax.slice(z8, (r0, 0), (r0 + PR, z8.shape[1]))
        zq = jax.lax.slice(z, (r0, 0), (r0 + PR, z.shape[1]))
        xp_ = jax.lax.slice(xb, (r0, 0), (r0 + PR, xb.shape[1]))
        logits = jax.lax.dot_general(
            zp, zq, (((1,), (1,)), ((), ())),
            preferred_element_type=jnp.float32)
        logits = jnp.where(cross, neg, logits)
        e = jnp.exp(logits)
        s0 = jnp.sum(e, axis=0, keepdims=True)
        inv_row = 1.0 / s0
        inv_col = jnp.reshape(inv_row, (PR, 1))
        adj = e * inv_col
        adj_ref[pl.ds(r0, NPG), :] = jax.lax.slice(
            adj, (0, 0), (NPG, NPG))
        adj_ref[pl.ds(r0 + NPG, NPG), :] = jax.lax.slice(
            adj, (NPG, NPG), (PR, PR))
        mask_row = 1.0 + jnp.sum(adj, axis=0, keepdims=True)
        e_dead = jnp.where(dead, 0.0, e)
        a0m = e_dead * (inv_row * mask_row)
        agg = jnp.dot(a0m, xp_,
                      preferred_element_type=jnp.float32)
        h = jnp.dot(agg, wp_ref[...], preferred_element_type=jnp.float32)
        h = jnp.maximum(h + bp_ref[...], 0.0)
        hg_rows.append(jnp.sum(
            jax.lax.slice(h, (0, 0), (NPG, h.shape[1])), axis=0,
            keepdims=True))
        hg_rows.append(jnp.sum(
            jax.lax.slice(h, (NPG, 0), (PR, h.shape[1])), axis=0,
            keepdims=True))

    hg = jnp.concatenate(hg_rows, axis=0)
    out_ref[...] = (
        jnp.dot(hg, wc_ref[...], preferred_element_type=jnp.float32)
        + bc_ref[...])


@functools.partial(jax.jit, static_argnames=("interpret",))
def _run(x, W_enc, b_enc, W_proc, b_proc, W_cls, b_cls, interpret=False):
    N, D = x.shape
    Bv = N // NPG
    H = W_enc.shape[1]
    C = W_cls.shape[1]
    grid = (Bv // G,)
    blk = G * NPG

    out, adj_flat = pl.pallas_call(
        _body,
        grid=grid,
        in_specs=[
            pl.BlockSpec((blk, D), lambda i: (i, 0)),
            pl.BlockSpec((D, H), lambda i: (0, 0)),
            pl.BlockSpec((1, H), lambda i: (0, 0)),
            pl.BlockSpec((D, H), lambda i: (0, 0)),
            pl.BlockSpec((1, H), lambda i: (0, 0)),
            pl.BlockSpec((H, C), lambda i: (0, 0)),
            pl.BlockSpec((1, C), lambda i: (0, 0)),
        ],
        out_specs=[
            pl.BlockSpec((G, C), lambda i: (i, 0)),
            pl.BlockSpec((blk, NPG), lambda i: (i, 0)),
        ],
        out_shape=[
            jax.ShapeDtypeStruct((Bv, C), jnp.float32),
            jax.ShapeDtypeStruct((N, NPG), jnp.float32),
        ],
        interpret=interpret,
    )(x, W_enc, b_enc.reshape(1, H), W_proc, b_proc.reshape(1, H),
      W_cls, b_cls.reshape(1, C))

    return out, adj_flat.reshape(Bv, NPG, NPG)


def kernel(x, edge_index, batch, W_enc, b_enc, W_proc, b_proc, W_cls, b_cls):
    return _run(x, W_enc, b_enc, W_proc, b_proc, W_cls, b_cls)


# speedup vs baseline: 2.0138x; 1.1603x over previous
import functools

import jax
import jax.numpy as jnp
from jax.experimental import pallas as pl

NPG = 61
P = 2
G = 32
PR = 2 * NPG


def _body(x_ref, we_ref, be_ref, wp_ref, bp_ref, wc_ref, bc_ref,
          out_ref, adj_ref):
    xb = x_ref[...]
    z = jnp.dot(xb, we_ref[...], preferred_element_type=jnp.float32)
    z = jnp.maximum(z + be_ref[...], 0.0)

    z8 = z * 0.125

    rows = jax.lax.broadcasted_iota(jnp.int32, (PR, PR), 0)
    cols = jax.lax.broadcasted_iota(jnp.int32, (PR, PR), 1)
    cross = (rows // NPG) != (cols // NPG)
    dead = cross | (rows == cols)
    neg = jnp.float32(-1e30)

    hg_rows = []
    for p in range(G // P):
        r0 = p * PR
        zp = jax.lax.slice(z8, (r0, 0), (r0 + PR, z8.shape[1]))
        zq = jax.lax.slice(z, (r0, 0), (r0 + PR, z.shape[1]))
        xp_ = jax.lax.slice(xb, (r0, 0), (r0 + PR, xb.shape[1]))
        logits = jax.lax.dot_general(
            zp, zq, (((1,), (1,)), ((), ())),
            preferred_element_type=jnp.float32)
        logits = jnp.where(cross, neg, logits)
        e = jnp.exp(logits)
        s0 = jnp.sum(e, axis=0, keepdims=True)
        inv_row = 1.0 / s0
        inv_col = jnp.reshape(inv_row, (PR, 1))
        adj = e * inv_col
        adj_ref[2 * p] = jax.lax.slice(adj, (0, 0), (NPG, NPG))
        adj_ref[2 * p + 1] = jax.lax.slice(adj, (NPG, NPG), (PR, PR))
        mask_row = 1.0 + jnp.sum(adj, axis=0, keepdims=True)
        e_dead = jnp.where(dead, 0.0, e)
        a0m = e_dead * (inv_row * mask_row)
        agg = jnp.dot(a0m, xp_,
                      preferred_element_type=jnp.float32)
        h = jnp.dot(agg, wp_ref[...], preferred_element_type=jnp.float32)
        h = jnp.maximum(h + bp_ref[...], 0.0)
        hg_rows.append(jnp.sum(
            jax.lax.slice(h, (0, 0), (NPG, h.shape[1])), axis=0,
            keepdims=True))
        hg_rows.append(jnp.sum(
            jax.lax.slice(h, (NPG, 0), (PR, h.shape[1])), axis=0,
            keepdims=True))

    hg = jnp.concatenate(hg_rows, axis=0)
    out_ref[...] = (
        jnp.dot(hg, wc_ref[...], preferred_element_type=jnp.float32)
        + bc_ref[...])


@functools.partial(jax.jit, static_argnames=("interpret",))
def _run(x, W_enc, b_enc, W_proc, b_proc, W_cls, b_cls, interpret=False):
    N, D = x.shape
    Bv = N // NPG
    H = W_enc.shape[1]
    C = W_cls.shape[1]
    grid = (Bv // G,)
    blk = G * NPG

    out, adj_flat = pl.pallas_call(
        _body,
        grid=grid,
        in_specs=[
            pl.BlockSpec((blk, D), lambda i: (i, 0)),
            pl.BlockSpec((D, H), lambda i: (0, 0)),
            pl.BlockSpec((1, H), lambda i: (0, 0)),
            pl.BlockSpec((D, H), lambda i: (0, 0)),
            pl.BlockSpec((1, H), lambda i: (0, 0)),
            pl.BlockSpec((H, C), lambda i: (0, 0)),
            pl.BlockSpec((1, C), lambda i: (0, 0)),
        ],
        out_specs=[
            pl.BlockSpec((G, C), lambda i: (i, 0)),
            pl.BlockSpec((G, NPG, NPG), lambda i: (i, 0, 0)),
        ],
        out_shape=[
            jax.ShapeDtypeStruct((Bv, C), jnp.float32),
            jax.ShapeDtypeStruct((Bv, NPG, NPG), jnp.float32),
        ],
        interpret=interpret,
    )(x, W_enc, b_enc.reshape(1, H), W_proc, b_proc.reshape(1, H),
      W_cls, b_cls.reshape(1, C))

    return out, adj_flat


def kernel(x, edge_index, batch, W_enc, b_enc, W_proc, b_proc, W_cls, b_cls):
    return _run(x, W_enc, b_enc, W_proc, b_proc, W_cls, b_cls)
